# baseline (speedup 1.0000x reference)
import functools

import jax
import jax.numpy as jnp
from jax import lax
from jax.experimental import pallas as pl
from jax.experimental.pallas import tpu as pltpu
from jax.experimental.pallas import tpu_sc as plsc

_BATCH = 16384
_DIM = 32
_NUM_CORES = 2
_NUM_SUBCORES = 16
_NW = _NUM_CORES * _NUM_SUBCORES
_BPW = _BATCH // _NW

_mesh = plsc.VectorSubcoreMesh(core_axis_name="c", subcore_axis_name="s")


@functools.partial(
    pl.kernel,
    mesh=_mesh,
    out_type=jax.ShapeDtypeStruct((_BATCH,), jnp.float32),
    compiler_params=pltpu.CompilerParams(
        needs_layout_passes=False, use_tc_tiling_on_sc=False),
    scratch_types=[
        pltpu.VMEM((_BPW,), jnp.int32),
        pltpu.VMEM((_BPW,), jnp.int32),
        pltpu.VMEM((_BPW,), jnp.int32),
        pltpu.VMEM((_DIM, _BPW), jnp.float32),
        pltpu.VMEM((_DIM, _BPW), jnp.float32),
        pltpu.VMEM((_DIM, _BPW), jnp.float32),
        pltpu.VMEM((_BPW,), jnp.float32),
        pltpu.SemaphoreType.DMA,
    ],
)
def _distmult_sc(tuples_hbm, nodeT_hbm, relaT_hbm, out_hbm,
                 hidx, tidx, ridx, hbuf, tbuf, rbuf, outv, sem):
    wid = lax.axis_index("s") * _NUM_CORES + lax.axis_index("c")
    base = wid * _BPW

    pltpu.sync_copy(tuples_hbm.at[pl.ds(base, _BPW)], hidx)
    pltpu.sync_copy(tuples_hbm.at[pl.ds(_BATCH + base, _BPW)], tidx)
    pltpu.sync_copy(tuples_hbm.at[pl.ds(2 * _BATCH + base, _BPW)], ridx)

    copies = []
    for d in range(_DIM):
        s = pl.ds(d * 1000000, 1000000)
        copies.append(pltpu.async_copy(nodeT_hbm.at[s].at[hidx], hbuf.at[d], sem))
        copies.append(pltpu.async_copy(nodeT_hbm.at[s].at[tidx], tbuf.at[d], sem))
        copies.append(pltpu.async_copy(relaT_hbm.at[s].at[ridx], rbuf.at[d], sem))
    for c in copies:
        c.wait()

    def group_body(g, carry):
        s = pl.ds(g * 16, 16)
        acc = jnp.zeros((16,), jnp.float32)
        for d in range(_DIM):
            acc = acc + hbuf[d, s] * rbuf[d, s] * tbuf[d, s]
        outv[s] = acc
        return carry

    lax.fori_loop(0, _BPW // 16, group_body, 0)

    pltpu.sync_copy(outv, out_hbm.at[pl.ds(base, _BPW)])


def kernel(tuples, node_emb, rela_emb):
    return _distmult_sc(tuples.reshape(-1), node_emb.T.reshape(-1),
                        rela_emb.T.reshape(-1))


# speedup vs baseline: 5.6158x; 5.6158x over previous
import functools

import jax
import jax.numpy as jnp
from jax import lax
from jax.experimental import pallas as pl
from jax.experimental.pallas import tpu as pltpu
from jax.experimental.pallas import tpu_sc as plsc

_BATCH = 16384
_DIM = 32
_NUM_CORES = 2
_NUM_SUBCORES = 16
_NW = _NUM_CORES * _NUM_SUBCORES
_BPW = _BATCH // _NW
_IDX_CHUNK = 128
_NCHUNK = _BPW // _IDX_CHUNK

_mesh = plsc.VectorSubcoreMesh(core_axis_name="c", subcore_axis_name="s")


@functools.partial(
    pl.kernel,
    mesh=_mesh,
    out_type=jax.ShapeDtypeStruct((_BATCH,), jnp.float32),
    compiler_params=pltpu.CompilerParams(
        needs_layout_passes=False, use_tc_tiling_on_sc=False),
    scratch_types=[
        pltpu.VMEM((_BPW,), jnp.int32),
        pltpu.VMEM((_BPW,), jnp.int32),
        pltpu.VMEM((_BPW,), jnp.int32),
        pltpu.VMEM((_BPW, _DIM), jnp.float32),
        pltpu.VMEM((_BPW, _DIM), jnp.float32),
        pltpu.VMEM((_BPW, _DIM), jnp.float32),
        pltpu.VMEM((_BPW,), jnp.float32),
        pltpu.SemaphoreType.DMA,
    ],
)
def _distmult_sc(tuples_hbm, node_hbm, rela_hbm, out_hbm,
                 hidx, tidx, ridx, hrows, trows, rrows, outv, sem):
    wid = lax.axis_index("s") * _NUM_CORES + lax.axis_index("c")
    base = wid * _BPW

    pltpu.sync_copy(tuples_hbm.at[pl.ds(base, _BPW)], hidx)
    pltpu.sync_copy(tuples_hbm.at[pl.ds(_BATCH + base, _BPW)], tidx)
    pltpu.sync_copy(tuples_hbm.at[pl.ds(2 * _BATCH + base, _BPW)], ridx)

    copies = []
    for j in range(_NCHUNK):
        s = pl.ds(j * _IDX_CHUNK, _IDX_CHUNK)
        copies.append(pltpu.async_copy(node_hbm.at[hidx.at[s]], hrows.at[s], sem))
        copies.append(pltpu.async_copy(node_hbm.at[tidx.at[s]], trows.at[s], sem))
        copies.append(pltpu.async_copy(rela_hbm.at[ridx.at[s]], rrows.at[s], sem))
    for c in copies:
        c.wait()

    def chunk_body(c, carry):
        rows = c * 16 + lax.iota(jnp.int32, 16)

        def d_body(d, acc):
            cols = jnp.full((16,), d, jnp.int32)
            hv = plsc.load_gather(hrows, [rows, cols])
            tv = plsc.load_gather(trows, [rows, cols])
            rv = plsc.load_gather(rrows, [rows, cols])
            return acc + hv * rv * tv

        acc = lax.fori_loop(0, _DIM, d_body, jnp.zeros((16,), jnp.float32))
        outv[pl.ds(c * 16, 16)] = acc
        return carry

    lax.fori_loop(0, _BPW // 16, chunk_body, 0)

    pltpu.sync_copy(outv, out_hbm.at[pl.ds(base, _BPW)])


def kernel(tuples, node_emb, rela_emb):
    return _distmult_sc(tuples.reshape(-1), node_emb, rela_emb)
